# baseline (speedup 1.0000x reference)
import functools

import jax
import jax.numpy as jnp
from jax import lax
from jax.experimental import pallas as pl
from jax.experimental.pallas import tpu as pltpu
from jax.experimental.pallas import tpu_sc as plsc

N_NODES = 10000
NPAD = 10240
NC = 2
NS = 16
NW = NC * NS
CH = 128
ROWS_PER_TILE = NPAD // NS
DEG_W = 16
BM = 1024


NB = 2
NI = 4
PF = 2


def _make_sc_agg(D, k, ka):
    mesh = plsc.VectorSubcoreMesh(core_axis_name="c", subcore_axis_name="s")
    assert k % 4 == 0 and k >= 8 and ka == k + PF

    scratch = (
        [pltpu.VMEM((CH,), jnp.int32) for _ in range(NI)]
        + [pltpu.VMEM((CH,), jnp.int32) for _ in range(NI)]
        + [pltpu.VMEM((CH, D), jnp.float32) for _ in range(NB)]
        + [pltpu.VMEM_SHARED((NPAD, D), jnp.float32)]
        + [pltpu.SemaphoreType.DMA for _ in range(NI + 2 * NB)]
    )

    @functools.partial(
        pl.kernel,
        mesh=mesh,
        out_type=jax.ShapeDtypeStruct((NC, NPAD, D), jnp.float32),
        scratch_types=scratch,
        compiler_params=pltpu.CompilerParams(use_tc_tiling_on_sc=False),
    )
    def agg(tab_hbm, src_hbm, dst_hbm, zeros_hbm, out_hbm, *rest):
        sidx = rest[:NI]
        didx = rest[NI:2 * NI]
        bufs = rest[2 * NI:2 * NI + NB]
        acc = rest[2 * NI + NB]
        sems = rest[2 * NI + NB + 1:]
        isem = sems[:NI]
        gsem = sems[NI:NI + NB]
        ssem = sems[NI + NB:]
        c = lax.axis_index("c")
        s = lax.axis_index("s")
        wid = c * NS + s
        r0 = s * ROWS_PER_TILE
        pltpu.sync_copy(zeros_hbm.at[pl.ds(r0, ROWS_PER_TILE)],
                        acc.at[pl.ds(r0, ROWS_PER_TILE)])
        plsc.subcore_barrier()

        def i_start(j, i4):
            pltpu.async_copy(src_hbm.at[wid, j], sidx[i4], isem[i4])
            pltpu.async_copy(dst_hbm.at[wid, j], didx[i4], isem[i4])

        def i_wait(j, i4):
            pltpu.make_async_copy(src_hbm.at[wid, j], sidx[i4],
                                  isem[i4]).wait()
            pltpu.make_async_copy(dst_hbm.at[wid, j], didx[i4],
                                  isem[i4]).wait()

        def g_start(b, i4):
            pltpu.async_copy(tab_hbm.at[sidx[i4]], bufs[b], gsem[b])

        def g_wait(b, i4):
            pltpu.make_async_copy(tab_hbm.at[sidx[i4]], bufs[b],
                                  gsem[b]).wait()

        def s_start(b, i4):
            pltpu.async_copy(bufs[b], acc.at[didx[i4]], ssem[b], add=True)

        def s_wait(b, i4):
            pltpu.make_async_copy(bufs[b], acc.at[didx[i4]],
                                  ssem[b]).wait()

        def slot(j, m4, *, skip_swait=False):
            b, b1 = m4 % 2, (m4 + 1) % 2
            g_wait(b, m4)
            s_start(b, m4)
            if not skip_swait:
                s_wait(b1, (m4 - 1) % NI)
            i_wait(j + 1, (m4 + 1) % NI)
            g_start(b1, (m4 + 1) % NI)
            i_start(j + 2, (m4 + 2) % NI)

        i_start(0, 0)
        i_start(1, 1)
        i_wait(0, 0)
        g_start(0, 0)
        slot(0, 0, skip_swait=True)
        slot(1, 1)

        def round_body(t, carry):
            j0 = t * 4 + 2
            for u in range(4):
                slot(j0 + u, (2 + u) % 4)
            return carry

        lax.fori_loop(0, (k - 4) // 4, round_body, 0)

        slot(k - 2, (k - 2) % 4)
        slot(k - 1, (k - 1) % 4)
        s_wait((k - 1) % 2, (k - 1) % NI)
        g_wait(k % 2, k % NI)
        i_wait(k + 1, (k + 1) % NI)
        plsc.subcore_barrier()
        pltpu.sync_copy(acc.at[pl.ds(r0, ROWS_PER_TILE)],
                        out_hbm.at[c, pl.ds(r0, ROWS_PER_TILE)])

    return agg


def _make_sc_deg(k):
    mesh = plsc.VectorSubcoreMesh(core_axis_name="c", subcore_axis_name="s")

    scratch = [
        pltpu.VMEM((k, CH), jnp.int32),
        pltpu.VMEM((CH, DEG_W), jnp.float32),
        pltpu.VMEM_SHARED((NPAD, DEG_W), jnp.float32),
        pltpu.SemaphoreType.DMA,
    ]

    @functools.partial(
        pl.kernel,
        mesh=mesh,
        out_type=jax.ShapeDtypeStruct((NC, NPAD, DEG_W), jnp.float32),
        scratch_types=scratch,
        compiler_params=pltpu.CompilerParams(use_tc_tiling_on_sc=False),
    )
    def deg(dst_hbm, zeros_hbm, out_hbm, didx_v, rows_v, acc, sem):
        c = lax.axis_index("c")
        s = lax.axis_index("s")
        wid = c * NS + s
        r0 = s * ROWS_PER_TILE
        pltpu.sync_copy(zeros_hbm.at[pl.ds(r0, ROWS_PER_TILE)],
                        acc.at[pl.ds(r0, ROWS_PER_TILE)])
        pltpu.sync_copy(dst_hbm.at[wid, pl.ds(0, k)], didx_v)
        ones = jnp.full((16,), 1.0, jnp.float32)
        for i in range(CH):
            rows_v[i, :] = ones
        plsc.subcore_barrier()

        def fire(j, carry):
            pltpu.async_copy(rows_v, acc.at[didx_v.at[j]], sem, add=True)
            return carry

        def drain(j, carry):
            pltpu.make_async_copy(rows_v, acc.at[didx_v.at[j]], sem).wait()
            return carry

        lax.fori_loop(0, k, fire, 0)
        lax.fori_loop(0, k, drain, 0)
        plsc.subcore_barrier()
        pltpu.sync_copy(acc.at[pl.ds(r0, ROWS_PER_TILE)],
                        out_hbm.at[c, pl.ds(r0, ROWS_PER_TILE)])

    return deg


def _tc_matmul(x, w):
    m, kdim = x.shape
    n = w.shape[1]

    def body(x_ref, w_ref, o_ref):
        o_ref[...] = jnp.dot(x_ref[...], w_ref[...],
                             preferred_element_type=jnp.float32)

    return pl.pallas_call(
        body,
        grid=(m // BM,),
        in_specs=[
            pl.BlockSpec((BM, kdim), lambda i: (i, 0)),
            pl.BlockSpec((kdim, n), lambda i: (0, 0)),
        ],
        out_specs=pl.BlockSpec((BM, n), lambda i: (i, 0)),
        out_shape=jax.ShapeDtypeStruct((m, n), jnp.float32),
    )(x, w)


def _tc_scale(h, d0, d1):
    m, n = h.shape

    def body(h_ref, d0_ref, d1_ref, o_ref):
        dinv = lax.rsqrt(d0_ref[...] + d1_ref[...] + 1.0)
        o_ref[...] = h_ref[...] * dinv

    return pl.pallas_call(
        body,
        grid=(m // BM,),
        in_specs=[
            pl.BlockSpec((BM, n), lambda i: (i, 0)),
            pl.BlockSpec((BM, 1), lambda i: (i, 0)),
            pl.BlockSpec((BM, 1), lambda i: (i, 0)),
        ],
        out_specs=pl.BlockSpec((BM, n), lambda i: (i, 0)),
        out_shape=jax.ShapeDtypeStruct((m, n), jnp.float32),
    )(h, d0, d1)


def _tc_mid(p0, p1, hs1, d0, d1, b1, w2):
    m, n = hs1.shape
    n2 = w2.shape[1]

    def body(p0_ref, p1_ref, hs1_ref, d0_ref, d1_ref, b1_ref, w2_ref, o_ref):
        dinv = lax.rsqrt(d0_ref[...] + d1_ref[...] + 1.0)
        out1 = dinv * (p0_ref[...] + p1_ref[...] + hs1_ref[...]) + b1_ref[...]
        a = jnp.maximum(out1, 0.0)
        o_ref[...] = dinv * jnp.dot(a, w2_ref[...],
                                    preferred_element_type=jnp.float32)

    return pl.pallas_call(
        body,
        grid=(m // BM,),
        in_specs=[
            pl.BlockSpec((BM, n), lambda i: (i, 0)),
            pl.BlockSpec((BM, n), lambda i: (i, 0)),
            pl.BlockSpec((BM, n), lambda i: (i, 0)),
            pl.BlockSpec((BM, 1), lambda i: (i, 0)),
            pl.BlockSpec((BM, 1), lambda i: (i, 0)),
            pl.BlockSpec((1, n), lambda i: (0, 0)),
            pl.BlockSpec((n, n2), lambda i: (0, 0)),
        ],
        out_specs=pl.BlockSpec((BM, n2), lambda i: (i, 0)),
        out_shape=jax.ShapeDtypeStruct((m, n2), jnp.float32),
    )(p0, p1, hs1, d0, d1, b1, w2)


def _tc_final(p0, p1, hs2, d0, d1, b2):
    m, n = hs2.shape

    def body(p0_ref, p1_ref, hs2_ref, d0_ref, d1_ref, b2_ref, o_ref):
        dinv = lax.rsqrt(d0_ref[...] + d1_ref[...] + 1.0)
        o_ref[...] = dinv * (p0_ref[...] + p1_ref[...] + hs2_ref[...]) + b2_ref[...]

    return pl.pallas_call(
        body,
        grid=(m // BM,),
        in_specs=[
            pl.BlockSpec((BM, n), lambda i: (i, 0)),
            pl.BlockSpec((BM, n), lambda i: (i, 0)),
            pl.BlockSpec((BM, n), lambda i: (i, 0)),
            pl.BlockSpec((BM, 1), lambda i: (i, 0)),
            pl.BlockSpec((BM, 1), lambda i: (i, 0)),
            pl.BlockSpec((1, n), lambda i: (0, 0)),
        ],
        out_specs=pl.BlockSpec((BM, n), lambda i: (i, 0)),
        out_shape=jax.ShapeDtypeStruct((m, n), jnp.float32),
    )(p0, p1, hs2, d0, d1, b2)


def kernel(x, edge_index, W1, b1, W2, b2):
    n, in_dim = x.shape
    hid = W1.shape[1]
    out_dim = W2.shape[1]
    e = edge_index.shape[1]
    k = -(-e // (NW * CH))
    k += (-k) % NB
    ka = k + PF
    epad = NW * CH * k

    src = edge_index[0].astype(jnp.int32)
    dst = edge_index[1].astype(jnp.int32)
    fill = jnp.full((epad - e,), n, jnp.int32)
    junk = jnp.full((NW, PF, CH), n, jnp.int32)
    src_p = jnp.concatenate(
        [jnp.concatenate([src, fill]).reshape(NW, k, CH), junk], axis=1)
    dst_p = jnp.concatenate(
        [jnp.concatenate([dst, fill]).reshape(NW, k, CH), junk], axis=1)

    x_p = jnp.pad(x, ((0, NPAD - n), (0, 0)))
    b1r = b1.reshape(1, hid)
    b2r = b2.reshape(1, out_dim)

    zeros_w = jnp.zeros((NPAD, DEG_W), jnp.float32)
    zeros_h = jnp.zeros((NPAD, hid), jnp.float32)
    zeros_o = jnp.zeros((NPAD, out_dim), jnp.float32)

    pdeg = _make_sc_deg(k)(dst_p, zeros_w)
    h1 = _tc_matmul(x_p, W1)

    d0 = pdeg[0, :, 0:1]
    d1 = pdeg[1, :, 0:1]

    hs1 = _tc_scale(h1, d0, d1)
    p1 = _make_sc_agg(hid, k, ka)(hs1, src_p, dst_p, zeros_h)

    hs2 = _tc_mid(p1[0], p1[1], hs1, d0, d1, b1r, W2)
    p2 = _make_sc_agg(out_dim, k, ka)(hs2, src_p, dst_p, zeros_o)

    z = _tc_final(p2[0], p2[1], hs2, d0, d1, b2r)
    return z[:n]


# speedup vs baseline: 1.2730x; 1.2730x over previous
import functools

import jax
import jax.numpy as jnp
from jax import lax
from jax.experimental import pallas as pl
from jax.experimental.pallas import tpu as pltpu
from jax.experimental.pallas import tpu_sc as plsc

N_NODES = 10000
NPAD = 10240
NC = 2
NS = 16
NW = NC * NS
CH = 128
ROWS_PER_TILE = NPAD // NS
DEG_W = 16
BM = 1024


RND = 16


def _make_sc_agg(D, k):
    mesh = plsc.VectorSubcoreMesh(core_axis_name="c", subcore_axis_name="s")
    assert k % RND == 0

    scratch = (
        [pltpu.VMEM((RND, CH), jnp.int32),
         pltpu.VMEM((RND, CH), jnp.int32)]
        + [pltpu.VMEM((CH, D), jnp.float32) for _ in range(2)]
        + [pltpu.VMEM_SHARED((NPAD, D), jnp.float32)]
        + [pltpu.SemaphoreType.DMA for _ in range(4)]
    )

    @functools.partial(
        pl.kernel,
        mesh=mesh,
        out_type=jax.ShapeDtypeStruct((NC, NPAD, D), jnp.float32),
        scratch_types=scratch,
        compiler_params=pltpu.CompilerParams(use_tc_tiling_on_sc=False),
    )
    def agg(tab_hbm, src_hbm, dst_hbm, zeros_hbm, out_hbm,
            sidx, didx, buf0, buf1, acc, g0, g1, s0, s1):
        bufs = (buf0, buf1)
        gsem = (g0, g1)
        ssem = (s0, s1)
        c = lax.axis_index("c")
        s = lax.axis_index("s")
        wid = c * NS + s
        r0 = s * ROWS_PER_TILE
        pltpu.sync_copy(zeros_hbm.at[pl.ds(r0, ROWS_PER_TILE)],
                        acc.at[pl.ds(r0, ROWS_PER_TILE)])
        plsc.subcore_barrier()

        def round_body(t, carry):
            j0 = t * RND
            pltpu.sync_copy(src_hbm.at[wid, pl.ds(j0, RND)], sidx)
            pltpu.sync_copy(dst_hbm.at[wid, pl.ds(j0, RND)], didx)
            gd = [None, None]
            sd = [None, None]
            gd[0] = pltpu.async_copy(tab_hbm.at[sidx.at[0]], bufs[0],
                                     gsem[0])
            for u in range(RND):
                b = u % 2
                gd[b].wait()
                sd[b] = pltpu.async_copy(bufs[b], acc.at[didx.at[u]],
                                         ssem[b], add=True)
                if u > 0:
                    sd[1 - b].wait()
                if u + 1 < RND:
                    gd[1 - b] = pltpu.async_copy(tab_hbm.at[sidx.at[u + 1]],
                                                 bufs[1 - b], gsem[1 - b])
            sd[(RND - 1) % 2].wait()
            return carry

        lax.fori_loop(0, k // RND, round_body, 0)
        plsc.subcore_barrier()
        pltpu.sync_copy(acc.at[pl.ds(r0, ROWS_PER_TILE)],
                        out_hbm.at[c, pl.ds(r0, ROWS_PER_TILE)])

    return agg


def _make_sc_deg(k):
    mesh = plsc.VectorSubcoreMesh(core_axis_name="c", subcore_axis_name="s")

    scratch = [
        pltpu.VMEM((k, CH), jnp.int32),
        pltpu.VMEM((CH, DEG_W), jnp.float32),
        pltpu.VMEM_SHARED((NPAD, DEG_W), jnp.float32),
        pltpu.SemaphoreType.DMA,
    ]

    @functools.partial(
        pl.kernel,
        mesh=mesh,
        out_type=jax.ShapeDtypeStruct((NC, NPAD, DEG_W), jnp.float32),
        scratch_types=scratch,
        compiler_params=pltpu.CompilerParams(use_tc_tiling_on_sc=False),
    )
    def deg(dst_hbm, zeros_hbm, out_hbm, didx_v, rows_v, acc, sem):
        c = lax.axis_index("c")
        s = lax.axis_index("s")
        wid = c * NS + s
        r0 = s * ROWS_PER_TILE
        pltpu.sync_copy(zeros_hbm.at[pl.ds(r0, ROWS_PER_TILE)],
                        acc.at[pl.ds(r0, ROWS_PER_TILE)])
        pltpu.sync_copy(dst_hbm.at[wid, pl.ds(0, k)], didx_v)
        ones = jnp.full((16,), 1.0, jnp.float32)
        for i in range(CH):
            rows_v[i, :] = ones
        plsc.subcore_barrier()

        def fire(j, carry):
            pltpu.async_copy(rows_v, acc.at[didx_v.at[j]], sem, add=True)
            return carry

        def drain(j, carry):
            pltpu.make_async_copy(rows_v, acc.at[didx_v.at[j]], sem).wait()
            return carry

        lax.fori_loop(0, k, fire, 0)
        lax.fori_loop(0, k, drain, 0)
        plsc.subcore_barrier()
        pltpu.sync_copy(acc.at[pl.ds(r0, ROWS_PER_TILE)],
                        out_hbm.at[c, pl.ds(r0, ROWS_PER_TILE)])

    return deg


def _tc_matmul(x, w):
    m, kdim = x.shape
    n = w.shape[1]

    def body(x_ref, w_ref, o_ref):
        o_ref[...] = jnp.dot(x_ref[...], w_ref[...],
                             preferred_element_type=jnp.float32)

    return pl.pallas_call(
        body,
        grid=(m // BM,),
        in_specs=[
            pl.BlockSpec((BM, kdim), lambda i: (i, 0)),
            pl.BlockSpec((kdim, n), lambda i: (0, 0)),
        ],
        out_specs=pl.BlockSpec((BM, n), lambda i: (i, 0)),
        out_shape=jax.ShapeDtypeStruct((m, n), jnp.float32),
    )(x, w)


def _tc_scale(h, d0, d1):
    m, n = h.shape

    def body(h_ref, d0_ref, d1_ref, o_ref):
        dinv = lax.rsqrt(d0_ref[...] + d1_ref[...] + 1.0)
        o_ref[...] = h_ref[...] * dinv

    return pl.pallas_call(
        body,
        grid=(m // BM,),
        in_specs=[
            pl.BlockSpec((BM, n), lambda i: (i, 0)),
            pl.BlockSpec((BM, 1), lambda i: (i, 0)),
            pl.BlockSpec((BM, 1), lambda i: (i, 0)),
        ],
        out_specs=pl.BlockSpec((BM, n), lambda i: (i, 0)),
        out_shape=jax.ShapeDtypeStruct((m, n), jnp.float32),
    )(h, d0, d1)


def _tc_mid(p0, p1, hs1, d0, d1, b1, w2):
    m, n = hs1.shape
    n2 = w2.shape[1]

    def body(p0_ref, p1_ref, hs1_ref, d0_ref, d1_ref, b1_ref, w2_ref, o_ref):
        dinv = lax.rsqrt(d0_ref[...] + d1_ref[...] + 1.0)
        out1 = dinv * (p0_ref[...] + p1_ref[...] + hs1_ref[...]) + b1_ref[...]
        a = jnp.maximum(out1, 0.0)
        o_ref[...] = dinv * jnp.dot(a, w2_ref[...],
                                    preferred_element_type=jnp.float32)

    return pl.pallas_call(
        body,
        grid=(m // BM,),
        in_specs=[
            pl.BlockSpec((BM, n), lambda i: (i, 0)),
            pl.BlockSpec((BM, n), lambda i: (i, 0)),
            pl.BlockSpec((BM, n), lambda i: (i, 0)),
            pl.BlockSpec((BM, 1), lambda i: (i, 0)),
            pl.BlockSpec((BM, 1), lambda i: (i, 0)),
            pl.BlockSpec((1, n), lambda i: (0, 0)),
            pl.BlockSpec((n, n2), lambda i: (0, 0)),
        ],
        out_specs=pl.BlockSpec((BM, n2), lambda i: (i, 0)),
        out_shape=jax.ShapeDtypeStruct((m, n2), jnp.float32),
    )(p0, p1, hs1, d0, d1, b1, w2)


def _tc_final(p0, p1, hs2, d0, d1, b2):
    m, n = hs2.shape

    def body(p0_ref, p1_ref, hs2_ref, d0_ref, d1_ref, b2_ref, o_ref):
        dinv = lax.rsqrt(d0_ref[...] + d1_ref[...] + 1.0)
        o_ref[...] = dinv * (p0_ref[...] + p1_ref[...] + hs2_ref[...]) + b2_ref[...]

    return pl.pallas_call(
        body,
        grid=(m // BM,),
        in_specs=[
            pl.BlockSpec((BM, n), lambda i: (i, 0)),
            pl.BlockSpec((BM, n), lambda i: (i, 0)),
            pl.BlockSpec((BM, n), lambda i: (i, 0)),
            pl.BlockSpec((BM, 1), lambda i: (i, 0)),
            pl.BlockSpec((BM, 1), lambda i: (i, 0)),
            pl.BlockSpec((1, n), lambda i: (0, 0)),
        ],
        out_specs=pl.BlockSpec((BM, n), lambda i: (i, 0)),
        out_shape=jax.ShapeDtypeStruct((m, n), jnp.float32),
    )(p0, p1, hs2, d0, d1, b2)


def kernel(x, edge_index, W1, b1, W2, b2):
    n, in_dim = x.shape
    hid = W1.shape[1]
    out_dim = W2.shape[1]
    e = edge_index.shape[1]
    k = -(-e // (NW * CH))
    k += (-k) % RND
    epad = NW * CH * k

    src = edge_index[0].astype(jnp.int32)
    dst = edge_index[1].astype(jnp.int32)
    fill = jnp.full((epad - e,), n, jnp.int32)
    src_p = jnp.concatenate([src, fill]).reshape(NW, k, CH)
    dst_p = jnp.concatenate([dst, fill]).reshape(NW, k, CH)

    x_p = jnp.pad(x, ((0, NPAD - n), (0, 0)))
    b1r = b1.reshape(1, hid)
    b2r = b2.reshape(1, out_dim)

    zeros_w = jnp.zeros((NPAD, DEG_W), jnp.float32)
    zeros_h = jnp.zeros((NPAD, hid), jnp.float32)
    zeros_o = jnp.zeros((NPAD, out_dim), jnp.float32)

    pdeg = _make_sc_deg(k)(dst_p, zeros_w)
    h1 = _tc_matmul(x_p, W1)

    d0 = pdeg[0, :, 0:1]
    d1 = pdeg[1, :, 0:1]

    hs1 = _tc_scale(h1, d0, d1)
    p1 = _make_sc_agg(hid, k)(hs1, src_p, dst_p, zeros_h)

    hs2 = _tc_mid(p1[0], p1[1], hs1, d0, d1, b1r, W2)
    p2 = _make_sc_agg(out_dim, k)(hs2, src_p, dst_p, zeros_o)

    z = _tc_final(p2[0], p2[1], hs2, d0, d1, b2r)
    return z[:n]


# speedup vs baseline: 1.2984x; 1.0200x over previous
import functools

import jax
import jax.numpy as jnp
from jax import lax
from jax.experimental import pallas as pl
from jax.experimental.pallas import tpu as pltpu
from jax.experimental.pallas import tpu_sc as plsc

N_NODES = 10000
NPAD = 10240
NC = 2
NS = 16
NW = NC * NS
CH = 128
ROWS_PER_TILE = NPAD // NS
DEG_W = 16
BM = 1024


RND = 8


def _make_sc_agg(D, k):
    mesh = plsc.VectorSubcoreMesh(core_axis_name="c", subcore_axis_name="s")
    assert k % RND == 0

    scratch = (
        [pltpu.VMEM((RND, CH), jnp.int32),
         pltpu.VMEM((RND, CH), jnp.int32)]
        + [pltpu.VMEM((CH, D), jnp.float32) for _ in range(RND)]
        + [pltpu.VMEM_SHARED((NPAD, D), jnp.float32)]
        + [pltpu.SemaphoreType.DMA for _ in range(2 * RND)]
    )

    @functools.partial(
        pl.kernel,
        mesh=mesh,
        out_type=jax.ShapeDtypeStruct((NC, NPAD, D), jnp.float32),
        scratch_types=scratch,
        compiler_params=pltpu.CompilerParams(use_tc_tiling_on_sc=False),
    )
    def agg(tab_hbm, src_hbm, dst_hbm, zeros_hbm, out_hbm,
            sidx, didx, *rest):
        bufs = rest[:RND]
        acc = rest[RND]
        gsem = rest[RND + 1:2 * RND + 1]
        ssem = rest[2 * RND + 1:]
        c = lax.axis_index("c")
        s = lax.axis_index("s")
        wid = c * NS + s
        r0 = s * ROWS_PER_TILE
        pltpu.sync_copy(zeros_hbm.at[pl.ds(r0, ROWS_PER_TILE)],
                        acc.at[pl.ds(r0, ROWS_PER_TILE)])
        plsc.subcore_barrier()

        def round_body(t, carry):
            j0 = t * RND
            pltpu.sync_copy(src_hbm.at[wid, pl.ds(j0, RND)], sidx)
            pltpu.sync_copy(dst_hbm.at[wid, pl.ds(j0, RND)], didx)
            gd = [pltpu.async_copy(tab_hbm.at[sidx.at[u]], bufs[u], gsem[u])
                  for u in range(RND)]
            sd = []
            for u in range(RND):
                gd[u].wait()
                sd.append(pltpu.async_copy(bufs[u], acc.at[didx.at[u]],
                                           ssem[u], add=True))
            for u in range(RND):
                sd[u].wait()
            return carry

        lax.fori_loop(0, k // RND, round_body, 0)
        plsc.subcore_barrier()
        pltpu.sync_copy(acc.at[pl.ds(r0, ROWS_PER_TILE)],
                        out_hbm.at[c, pl.ds(r0, ROWS_PER_TILE)])

    return agg


def _make_sc_deg(k):
    mesh = plsc.VectorSubcoreMesh(core_axis_name="c", subcore_axis_name="s")

    scratch = [
        pltpu.VMEM((k, CH), jnp.int32),
        pltpu.VMEM((CH, DEG_W), jnp.float32),
        pltpu.VMEM_SHARED((NPAD, DEG_W), jnp.float32),
        pltpu.SemaphoreType.DMA,
    ]

    @functools.partial(
        pl.kernel,
        mesh=mesh,
        out_type=jax.ShapeDtypeStruct((NC, NPAD, DEG_W), jnp.float32),
        scratch_types=scratch,
        compiler_params=pltpu.CompilerParams(use_tc_tiling_on_sc=False),
    )
    def deg(dst_hbm, zeros_hbm, out_hbm, didx_v, rows_v, acc, sem):
        c = lax.axis_index("c")
        s = lax.axis_index("s")
        wid = c * NS + s
        r0 = s * ROWS_PER_TILE
        pltpu.sync_copy(zeros_hbm.at[pl.ds(r0, ROWS_PER_TILE)],
                        acc.at[pl.ds(r0, ROWS_PER_TILE)])
        pltpu.sync_copy(dst_hbm.at[wid, pl.ds(0, k)], didx_v)
        ones = jnp.full((16,), 1.0, jnp.float32)
        for i in range(CH):
            rows_v[i, :] = ones
        plsc.subcore_barrier()

        def fire(j, carry):
            pltpu.async_copy(rows_v, acc.at[didx_v.at[j]], sem, add=True)
            return carry

        def drain(j, carry):
            pltpu.make_async_copy(rows_v, acc.at[didx_v.at[j]], sem).wait()
            return carry

        lax.fori_loop(0, k, fire, 0)
        lax.fori_loop(0, k, drain, 0)
        plsc.subcore_barrier()
        pltpu.sync_copy(acc.at[pl.ds(r0, ROWS_PER_TILE)],
                        out_hbm.at[c, pl.ds(r0, ROWS_PER_TILE)])

    return deg


def _tc_matmul(x, w):
    m, kdim = x.shape
    n = w.shape[1]

    def body(x_ref, w_ref, o_ref):
        o_ref[...] = jnp.dot(x_ref[...], w_ref[...],
                             preferred_element_type=jnp.float32)

    return pl.pallas_call(
        body,
        grid=(m // BM,),
        in_specs=[
            pl.BlockSpec((BM, kdim), lambda i: (i, 0)),
            pl.BlockSpec((kdim, n), lambda i: (0, 0)),
        ],
        out_specs=pl.BlockSpec((BM, n), lambda i: (i, 0)),
        out_shape=jax.ShapeDtypeStruct((m, n), jnp.float32),
    )(x, w)


def _tc_scale(h, d0, d1):
    m, n = h.shape
    hh = n // 2

    def body(h_ref, d0_ref, d1_ref, oa_ref, ob_ref):
        dinv = lax.rsqrt(d0_ref[...] + d1_ref[...] + 1.0)
        hs = h_ref[...] * dinv
        oa_ref[...] = hs[:, :hh]
        ob_ref[...] = hs[:, hh:]

    return pl.pallas_call(
        body,
        grid=(m // BM,),
        in_specs=[
            pl.BlockSpec((BM, n), lambda i: (i, 0)),
            pl.BlockSpec((BM, 1), lambda i: (i, 0)),
            pl.BlockSpec((BM, 1), lambda i: (i, 0)),
        ],
        out_specs=[
            pl.BlockSpec((BM, hh), lambda i: (i, 0)),
            pl.BlockSpec((BM, hh), lambda i: (i, 0)),
        ],
        out_shape=[
            jax.ShapeDtypeStruct((m, hh), jnp.float32),
            jax.ShapeDtypeStruct((m, hh), jnp.float32),
        ],
    )(h, d0, d1)


def _tc_mid(pa0, pa1, pb0, pb1, hsa, hsb, d0, d1, b1, w2):
    m, hh = hsa.shape
    n2 = w2.shape[1]

    def body(pa0_ref, pa1_ref, pb0_ref, pb1_ref, hsa_ref, hsb_ref,
             d0_ref, d1_ref, b1_ref, w2_ref, o_ref):
        dinv = lax.rsqrt(d0_ref[...] + d1_ref[...] + 1.0)
        outa = dinv * (pa0_ref[...] + pa1_ref[...] + hsa_ref[...])
        outb = dinv * (pb0_ref[...] + pb1_ref[...] + hsb_ref[...])
        out1 = jnp.concatenate([outa, outb], axis=1) + b1_ref[...]
        a = jnp.maximum(out1, 0.0)
        o_ref[...] = dinv * jnp.dot(a, w2_ref[...],
                                    preferred_element_type=jnp.float32)

    half = pl.BlockSpec((BM, hh), lambda i: (i, 0))
    col = pl.BlockSpec((BM, 1), lambda i: (i, 0))
    return pl.pallas_call(
        body,
        grid=(m // BM,),
        in_specs=[half, half, half, half, half, half, col, col,
                  pl.BlockSpec((1, 2 * hh), lambda i: (0, 0)),
                  pl.BlockSpec((2 * hh, n2), lambda i: (0, 0))],
        out_specs=pl.BlockSpec((BM, n2), lambda i: (i, 0)),
        out_shape=jax.ShapeDtypeStruct((m, n2), jnp.float32),
    )(pa0, pa1, pb0, pb1, hsa, hsb, d0, d1, b1, w2)


def _tc_final(p0, p1, hs2, d0, d1, b2):
    m, n = hs2.shape

    def body(p0_ref, p1_ref, hs2_ref, d0_ref, d1_ref, b2_ref, o_ref):
        dinv = lax.rsqrt(d0_ref[...] + d1_ref[...] + 1.0)
        o_ref[...] = dinv * (p0_ref[...] + p1_ref[...] + hs2_ref[...]) + b2_ref[...]

    return pl.pallas_call(
        body,
        grid=(m // BM,),
        in_specs=[
            pl.BlockSpec((BM, n), lambda i: (i, 0)),
            pl.BlockSpec((BM, n), lambda i: (i, 0)),
            pl.BlockSpec((BM, n), lambda i: (i, 0)),
            pl.BlockSpec((BM, 1), lambda i: (i, 0)),
            pl.BlockSpec((BM, 1), lambda i: (i, 0)),
            pl.BlockSpec((1, n), lambda i: (0, 0)),
        ],
        out_specs=pl.BlockSpec((BM, n), lambda i: (i, 0)),
        out_shape=jax.ShapeDtypeStruct((m, n), jnp.float32),
    )(p0, p1, hs2, d0, d1, b2)


def kernel(x, edge_index, W1, b1, W2, b2):
    n, in_dim = x.shape
    hid = W1.shape[1]
    out_dim = W2.shape[1]
    e = edge_index.shape[1]
    k = -(-e // (NW * CH))
    k += (-k) % RND
    epad = NW * CH * k

    src = edge_index[0].astype(jnp.int32)
    dst = edge_index[1].astype(jnp.int32)
    fill = jnp.full((epad - e,), n, jnp.int32)
    src_p = jnp.concatenate([src, fill]).reshape(NW, k, CH)
    dst_p = jnp.concatenate([dst, fill]).reshape(NW, k, CH)

    x_p = jnp.pad(x, ((0, NPAD - n), (0, 0)))
    b1r = b1.reshape(1, hid)
    b2r = b2.reshape(1, out_dim)

    zeros_w = jnp.zeros((NPAD, DEG_W), jnp.float32)
    zeros_h2 = jnp.zeros((NPAD, hid // 2), jnp.float32)
    zeros_o = jnp.zeros((NPAD, out_dim), jnp.float32)

    pdeg = _make_sc_deg(k)(dst_p, zeros_w)
    h1 = _tc_matmul(x_p, W1)

    d0 = pdeg[0, :, 0:1]
    d1 = pdeg[1, :, 0:1]

    hs1a, hs1b = _tc_scale(h1, d0, d1)
    agg64 = _make_sc_agg(hid // 2, k)
    p1a = agg64(hs1a, src_p, dst_p, zeros_h2)
    p1b = agg64(hs1b, src_p, dst_p, zeros_h2)

    hs2 = _tc_mid(p1a[0], p1a[1], p1b[0], p1b[1], hs1a, hs1b,
                  d0, d1, b1r, W2)
    p2 = _make_sc_agg(out_dim, k)(hs2, src_p, dst_p, zeros_o)

    z = _tc_final(p2[0], p2[1], hs2, d0, d1, b2r)
    return z[:n]
